# baseline (speedup 1.0000x reference)
import functools

import jax
import jax.numpy as jnp
from jax import lax
from jax.experimental import pallas as pl
from jax.experimental.pallas import tpu as pltpu
from jax.experimental.pallas import tpu_sc as plsc

TOKEN_LENGTH = 32
HIDDEN_SIZE = 128
BLOCK_ROWS = 512
NUM_WORKERS = 32


def _tc_body(x_ref, tab_ref, emb_ref):
    amp = x_ref[...].astype(jnp.float32) * 2.0 - 1.0
    tab = tab_ref[...]
    emb_ref[...] = amp[:, :, None] * tab[None, :, :]


def _tc_embeddings(x, emb_table):
    n, L = x.shape
    H = emb_table.shape[1]
    R = BLOCK_ROWS
    grid = (n // R,)
    return pl.pallas_call(
        _tc_body,
        grid=grid,
        in_specs=[
            pl.BlockSpec((R, L), lambda i: (i, 0)),
            pl.BlockSpec((L, H), lambda i: (0, 0)),
        ],
        out_specs=pl.BlockSpec((R, L, H), lambda i: (i, 0, 0)),
        out_shape=jax.ShapeDtypeStruct((n, L, H), jnp.float32),
    )(x, emb_table)


def _sc_logit(x, emb_table_t):
    N, L = x.shape
    H = emb_table_t.shape[0]
    rows_per = N // NUM_WORKERS
    mesh = plsc.VectorSubcoreMesh(core_axis_name="c", subcore_axis_name="s")

    @functools.partial(
        pl.kernel,
        out_type=jax.ShapeDtypeStruct((N, L), jnp.float32),
        mesh=mesh,
        scratch_types=[
            pltpu.VMEM((H, L), jnp.float32),
            pltpu.VMEM((rows_per, L), jnp.int32),
            pltpu.VMEM((rows_per, L), jnp.float32),
        ],
    )
    def sc_kernel(x_hbm, tabt_hbm, out_hbm, tabt_v, x_v, out_v):
        wid = lax.axis_index("s") * 2 + lax.axis_index("c")
        base = wid * rows_per
        pltpu.sync_copy(tabt_hbm, tabt_v)
        pltpu.sync_copy(x_hbm.at[pl.ds(base, rows_per)], x_v)

        rs = []
        for j in range(2):
            acc = jnp.zeros((16,), jnp.float32)
            for h in range(H):
                acc = acc + tabt_v[h, pl.ds(16 * j, 16)]
            rs.append(acc)

        def body(i, carry):
            for j in range(2):
                xv = x_v[i, pl.ds(16 * j, 16)]
                amp = xv.astype(jnp.float32) * 2.0 - 1.0
                out_v[i, pl.ds(16 * j, 16)] = amp * rs[j]
            return carry

        lax.fori_loop(0, rows_per, body, 0)
        pltpu.sync_copy(out_v, out_hbm.at[pl.ds(base, rows_per)])

    return sc_kernel(x, emb_table_t)


def kernel(text_batch, emb_table):
    B, S, L = text_batch.shape
    H = emb_table.shape[1]
    N = B * S
    x = text_batch.reshape(N, L)
    emb_flat = _tc_embeddings(x, emb_table)
    logit_flat = _sc_logit(x, emb_table.T)
    embeddings = emb_flat.reshape(B, S, L, H)
    logit_prime = logit_flat.reshape(B, S, L, 1)
    return embeddings, logit_prime


# speedup vs baseline: 1.3543x; 1.3543x over previous
import jax
import jax.numpy as jnp
from jax.experimental import pallas as pl

TOKEN_LENGTH = 32
HIDDEN_SIZE = 128
BLOCK_ROWS = 512


def _body(x_ref, tab_ref, emb_ref, logit_ref):
    amp = x_ref[...].astype(jnp.float32) * 2.0 - 1.0
    tab = tab_ref[...]
    emb_ref[...] = amp[:, :, None] * tab[None, :, :]
    rowsum = jnp.sum(tab, axis=1)
    logit_ref[...] = amp * rowsum[None, :]


def kernel(text_batch, emb_table):
    B, S, L = text_batch.shape
    H = emb_table.shape[1]
    N = B * S
    x = text_batch.reshape(N, L)
    R = BLOCK_ROWS
    grid = (N // R,)
    emb_flat, logit_flat = pl.pallas_call(
        _body,
        grid=grid,
        in_specs=[
            pl.BlockSpec((R, L), lambda i: (i, 0)),
            pl.BlockSpec((L, H), lambda i: (0, 0)),
        ],
        out_specs=[
            pl.BlockSpec((R, L, H), lambda i: (i, 0, 0)),
            pl.BlockSpec((R, L), lambda i: (i, 0)),
        ],
        out_shape=[
            jax.ShapeDtypeStruct((N, L, H), jnp.float32),
            jax.ShapeDtypeStruct((N, L), jnp.float32),
        ],
    )(x, emb_table)
    embeddings = emb_flat.reshape(B, S, L, H)
    logit_prime = logit_flat.reshape(B, S, L, 1)
    return embeddings, logit_prime
